# baseline (speedup 1.0000x reference)
import dataclasses
import functools

import jax
import jax.numpy as jnp
from jax import lax
from jax.experimental import pallas as pl
from jax.experimental.pallas import tpu as pltpu
from jax.experimental.pallas import tpu_sc as plsc

NUM_CORES = 2
NUM_SUBCORES = 16
LANES = 16
NUM_WORKERS = NUM_CORES * NUM_SUBCORES

BATCH = 16384
EMBED = 128
BPW = BATCH // NUM_WORKERS
CHUNK = 128
NCHUNKS = BPW // CHUNK


def _dot_sigmoid_kernel(in_hbm, temb_hbm, nemb_hbm, tb_hbm, nb_hbm, out_hbm,
                        pair_v, tidx_v, nidx_v,
                        trows0, nrows0, tb0, nb0,
                        trows1, nrows1, tb1, nb1,
                        out_v, sem0, sem1):
  wid = lax.axis_index("s") * NUM_CORES + lax.axis_index("c")
  base = wid * BPW

  pltpu.sync_copy(in_hbm.at[pl.ds(2 * base, 2 * BPW)], pair_v)
  lane2 = lax.iota(jnp.int32, LANES) * 2

  @pl.loop(0, BPW // LANES)
  def _(g):
    ids = lane2 + g * (2 * LANES)
    tidx_v[pl.ds(g * LANES, LANES)] = plsc.load_gather(pair_v, [ids])
    nidx_v[pl.ds(g * LANES, LANES)] = plsc.load_gather(pair_v, [ids + 1])

  bufs = [(trows0, nrows0, tb0, nb0), (trows1, nrows1, tb1, nb1)]
  sems = [sem0, sem1]
  lane = lax.iota(jnp.int32, LANES)

  def fire(c, b, sem):
    i_t = tidx_v.at[pl.ds(c * CHUNK, CHUNK)]
    i_n = nidx_v.at[pl.ds(c * CHUNK, CHUNK)]
    trows, nrows, tbv, nbv = b
    return (pltpu.async_copy(temb_hbm.at[i_t], trows, sem),
            pltpu.async_copy(nemb_hbm.at[i_n], nrows, sem),
            pltpu.async_copy(tb_hbm.at[i_t], tbv, sem),
            pltpu.async_copy(nb_hbm.at[i_n], nbv, sem))

  inflight = fire(0, bufs[0], sems[0])
  for c in range(NCHUNKS):
    for cp in inflight:
      cp.wait()
    if c + 1 < NCHUNKS:
      inflight = fire(c + 1, bufs[(c + 1) % 2], sems[(c + 1) % 2])
    trows, nrows, tbv, nbv = bufs[c % 2]

    @pl.loop(0, CHUNK // LANES)
    def _(g, c=c, trows=trows, nrows=nrows, tbv=tbv, nbv=nbv):
      dots = jnp.zeros((LANES,), jnp.float32)
      for r in range(LANES):
        row = g * LANES + r
        acc = trows[row, pl.ds(0, LANES)] * nrows[row, pl.ds(0, LANES)]
        for k in range(1, EMBED // LANES):
          acc = acc + (trows[row, pl.ds(k * LANES, LANES)] *
                       nrows[row, pl.ds(k * LANES, LANES)])
        dots = jnp.where(lane == r, jnp.sum(acc), dots)
      xv = dots + tbv[pl.ds(g * LANES, LANES)] + nbv[pl.ds(g * LANES, LANES)]
      yv = 1.0 / (1.0 + jnp.exp(-xv))
      out_v[pl.ds(c * CHUNK + g * LANES, LANES)] = yv

  pltpu.sync_copy(out_v, out_hbm.at[pl.ds(base, BPW)])


@jax.jit
def _run(inputs, temb, nemb, tb, nb):
  mesh = plsc.VectorSubcoreMesh(core_axis_name="c", subcore_axis_name="s")
  cp = pltpu.CompilerParams()
  if "needs_layout_passes" in pltpu.CompilerParams.__dataclass_fields__:
    cp = dataclasses.replace(cp, needs_layout_passes=False)
  row_bufs = [pltpu.VMEM((CHUNK, EMBED), jnp.float32),
              pltpu.VMEM((CHUNK, EMBED), jnp.float32),
              pltpu.VMEM((CHUNK,), jnp.float32),
              pltpu.VMEM((CHUNK,), jnp.float32)]
  kern = pl.kernel(
      _dot_sigmoid_kernel,
      out_type=jax.ShapeDtypeStruct((BATCH,), jnp.float32),
      mesh=mesh,
      scratch_types=(
          [pltpu.VMEM((2 * BPW,), jnp.int32),
           pltpu.VMEM((BPW,), jnp.int32),
           pltpu.VMEM((BPW,), jnp.int32)]
          + row_bufs + row_bufs
          + [pltpu.VMEM((BPW,), jnp.float32),
             pltpu.SemaphoreType.DMA,
             pltpu.SemaphoreType.DMA]
      ),
      compiler_params=cp,
  )
  return kern(inputs, temb, nemb, tb, nb)


def kernel(inputs, track_embedding, name_embedding, track_bias, name_bias):
  tb = track_bias.reshape(-1)
  nb = name_bias.reshape(-1)
  flat = inputs.astype(jnp.int32).reshape(-1)
  return _run(flat, track_embedding, name_embedding, tb, nb)
